# baseline (speedup 1.0000x reference)
import functools

import jax
import jax.numpy as jnp
from jax import lax
from jax.experimental import pallas as pl
from jax.experimental.pallas import tpu as pltpu
from jax.experimental.pallas import tpu_sc as plsc

_VOCAB = 1000000
_D = 64
_PAD = 0

_B_ROWS = 4096
_SEQ = 200
_B = _B_ROWS * _SEQ

_NC = 2
_NS = 16
_NW = _NC * _NS
_IDX_W = 128
_GPC = 2
_CHUNK = _IDX_W * _GPC
_PER_W = _B // _NW
_N_GROUPS_W = _PER_W // _IDX_W
_N_CHUNKS = _N_GROUPS_W // _GPC
_BCOL = _B_ROWS // _IDX_W
_NBUF = 2
_DT = _D // 8


def _gather_kernel(idx_hbm, table_hbm, out_hbm, idx_v, mbufs, nbufs, gsems, osems):
    wid = lax.axis_index("s") * _NC + lax.axis_index("c")
    g0 = wid * _N_GROUPS_W
    iota = lax.iota(jnp.int32, 16)
    pltpu.sync_copy(idx_hbm.at[wid], idx_v)

    def gather_parts(t, b):
        return [
            (
                table_hbm.at[idx_v.at[t * _GPC + j]],
                mbufs[b].at[pl.ds(j * _IDX_W, _IDX_W)],
                gsems[b],
            )
            for j in range(_GPC)
        ]

    def wb_parts(t, b):
        parts = []
        for j in range(_GPC):
            grp = g0 + t * _GPC + j
            s = grp // _BCOL
            c = grp % _BCOL
            for g in range(_DT):
                parts.append(
                    (
                        nbufs[b].at[j, pl.ds(g * 8, 8)],
                        out_hbm.at[s, g, c],
                        osems[b],
                    )
                )
        return parts

    def transpose_chunk(b):
        for j in range(_GPC):
            base = j * _IDX_W

            @pl.loop(0, _D)
            def _(d):
                dv = jnp.full((16,), d, jnp.int32)
                for jj in range(_IDX_W // 16):
                    v = plsc.load_gather(
                        mbufs[b], [base + jj * 16 + iota, dv]
                    )
                    nbufs[b][j, d, pl.ds(jj * 16, 16)] = v

    for p in gather_parts(0, 0):
        pltpu.async_copy(*p)

    @pl.loop(0, _N_CHUNKS // _NBUF)
    def _(t2):
        for b in range(_NBUF):
            t = _NBUF * t2 + b
            for p in gather_parts(t, b):
                pltpu.make_async_copy(*p).wait()

            @pl.when(t + 1 < _N_CHUNKS)
            def _():
                for p in gather_parts(t + 1, (b + 1) % _NBUF):
                    pltpu.async_copy(*p)

            @pl.when(t >= _NBUF)
            def _():
                for p in wb_parts(t - _NBUF, b):
                    pltpu.make_async_copy(*p).wait()

            transpose_chunk(b)
            for p in wb_parts(t, b):
                pltpu.async_copy(*p)

    for t in (_N_CHUNKS - 2, _N_CHUNKS - 1):
        for p in wb_parts(t, t % _NBUF):
            pltpu.make_async_copy(*p).wait()


def _sc_gather(idx3, table):
    mesh = plsc.VectorSubcoreMesh(core_axis_name="c", subcore_axis_name="s")
    f = functools.partial(
        pl.kernel,
        out_type=jax.ShapeDtypeStruct((_SEQ, _DT, _BCOL, 8, _IDX_W), jnp.float32),
        mesh=mesh,
        scratch_types=[
            pltpu.VMEM((_N_GROUPS_W, _IDX_W), jnp.int32),
            [pltpu.VMEM((_CHUNK, _D), jnp.float32) for _ in range(_NBUF)],
            [pltpu.VMEM((_GPC, _D, _IDX_W), jnp.float32) for _ in range(_NBUF)],
            [pltpu.SemaphoreType.DMA for _ in range(_NBUF)],
            [pltpu.SemaphoreType.DMA for _ in range(_NBUF)],
        ],
        compiler_params=pltpu.CompilerParams(
            use_tc_tiling_on_sc=False, needs_layout_passes=False
        ),
    )(_gather_kernel)
    return f(idx3, table)


def _mask_body(x_ref, o_ref):
    o_ref[...] = x_ref[...] == _PAD


def _tc_mask(x):
    return pl.pallas_call(
        _mask_body,
        out_shape=jax.ShapeDtypeStruct((_B_ROWS, _SEQ), jnp.bool_),
    )(x)


def kernel(x, table):
    idx3 = (
        x.reshape(_BCOL, _IDX_W, _SEQ)
        .transpose(2, 0, 1)
        .reshape(_NW, _N_GROUPS_W, _IDX_W)
    )
    out5 = _sc_gather(idx3, table)
    h = out5.transpose(2, 4, 0, 1, 3).reshape(_B_ROWS, _SEQ, _D)
    mask = _tc_mask(x)
    return (h, mask)


# speedup vs baseline: 1.3450x; 1.3450x over previous
import functools

import jax
import jax.numpy as jnp
from jax import lax
from jax.experimental import pallas as pl
from jax.experimental.pallas import tpu as pltpu
from jax.experimental.pallas import tpu_sc as plsc

_VOCAB = 1000000
_D = 64
_PAD = 0

_B_ROWS = 4096
_SEQ = 200
_B = _B_ROWS * _SEQ

_NC = 2
_NS = 16
_NW = _NC * _NS
_IDX_W = 128
_GPC = 4
_CHUNK = _IDX_W * _GPC
_PER_W = _B // _NW
_N_IDX_ROWS = _PER_W // _IDX_W
_N_CHUNKS = _PER_W // _CHUNK
_NBUF = 2

_NGROUPS = _B // _IDX_W
_BCOL = _B_ROWS // _IDX_W
_GPS = 8


def _gather_kernel(idx_hbm, table_hbm, out_hbm, idx_v, bufs, gsems, osems):
    wid = lax.axis_index("s") * _NC + lax.axis_index("c")
    base = wid * _PER_W
    pltpu.sync_copy(idx_hbm.at[wid], idx_v)

    def fire_gathers(t, b):
        for j in range(_GPC):
            pltpu.async_copy(
                table_hbm.at[idx_v.at[t * _GPC + j]],
                bufs[b].at[pl.ds(j * _IDX_W, _IDX_W)],
                gsems[b],
            )

    def wait_gathers(t, b):
        for j in range(_GPC):
            pltpu.make_async_copy(
                table_hbm.at[idx_v.at[t * _GPC + j]],
                bufs[b].at[pl.ds(j * _IDX_W, _IDX_W)],
                gsems[b],
            ).wait()

    def out_copy(t, b):
        return (
            bufs[b],
            out_hbm.at[pl.ds(base + t * _CHUNK, _CHUNK)],
            osems[b],
        )

    fire_gathers(0, 0)

    @pl.loop(0, _N_CHUNKS // _NBUF)
    def _(t2):
        for b in range(_NBUF):
            t = _NBUF * t2 + b
            wait_gathers(t, b)
            nb = (b + 1) % _NBUF

            @pl.when(jnp.logical_and(t >= 1, t + 1 < _N_CHUNKS))
            def _():
                pltpu.make_async_copy(*out_copy(t - 1, nb)).wait()

            @pl.when(t + 1 < _N_CHUNKS)
            def _():
                fire_gathers(t + 1, nb)

            pltpu.async_copy(*out_copy(t, b))

    pltpu.make_async_copy(*out_copy(_N_CHUNKS - 2, (_N_CHUNKS - 2) % _NBUF)).wait()
    pltpu.make_async_copy(*out_copy(_N_CHUNKS - 1, (_N_CHUNKS - 1) % _NBUF)).wait()


def _sc_gather(idx3, table):
    mesh = plsc.VectorSubcoreMesh(core_axis_name="c", subcore_axis_name="s")
    f = functools.partial(
        pl.kernel,
        out_type=jax.ShapeDtypeStruct((_B, _D), jnp.float32),
        mesh=mesh,
        scratch_types=[
            pltpu.VMEM((_N_IDX_ROWS, _IDX_W), jnp.int32),
            [pltpu.VMEM((_CHUNK, _D), jnp.float32) for _ in range(_NBUF)],
            [pltpu.SemaphoreType.DMA for _ in range(_NBUF)],
            [pltpu.SemaphoreType.DMA for _ in range(_NBUF)],
        ],
        compiler_params=pltpu.CompilerParams(use_tc_tiling_on_sc=False),
    )(_gather_kernel)
    return f(idx3, table)


def _shuffle_body(g_ref, o_ref):
    tj = lax.broadcasted_iota(jnp.int32, (_D, _IDX_W), 0)
    jj = lax.broadcasted_iota(jnp.int32, (_D, _IDX_W), 1)
    e0 = (jj == 2 * tj).astype(jnp.float32)
    e1 = (jj == 2 * tj + 1).astype(jnp.float32)
    cn = (((0,), (0,)), ((), ()))
    outs = []
    for k in range(_GPS):
        m2 = g_ref[pl.ds(k * 8, 8), :, :].reshape(_D, _IDX_W)
        a = m2[:, 0:_D]
        b = m2[:, _D:_IDX_W]
        n = lax.dot_general(a, e0, cn, preferred_element_type=jnp.float32)
        n += lax.dot_general(b, e1, cn, preferred_element_type=jnp.float32)
        outs.append(n)
    o_ref[...] = jnp.concatenate(outs, axis=1).reshape(1, _D, _GPS * _IDX_W)


def _tc_shuffle(g):
    return pl.pallas_call(
        _shuffle_body,
        grid=(_NGROUPS // _GPS,),
        in_specs=[pl.BlockSpec((8 * _GPS, 8, 128), lambda i: (i, 0, 0))],
        out_specs=pl.BlockSpec(
            (1, _D, _GPS * _IDX_W),
            lambda i: (i // (_BCOL // _GPS), 0, i % (_BCOL // _GPS)),
        ),
        out_shape=jax.ShapeDtypeStruct((_SEQ, _D, _B_ROWS), jnp.float32),
    )(g)


def _mask_body(x_ref, o_ref):
    o_ref[...] = x_ref[...] == _PAD


def _tc_mask(x):
    return pl.pallas_call(
        _mask_body,
        out_shape=jax.ShapeDtypeStruct((_B_ROWS, _SEQ), jnp.bool_),
    )(x)


def kernel(x, table):
    idx3 = (
        x.reshape(_BCOL, _IDX_W, _SEQ)
        .transpose(2, 0, 1)
        .reshape(_NW, _N_IDX_ROWS, _IDX_W)
    )
    g = _sc_gather(idx3, table).reshape(_NGROUPS * 8, 8, _IDX_W)
    hq = _tc_shuffle(g)
    h = hq.transpose(2, 0, 1)
    mask = _tc_mask(x)
    return (h, mask)


# speedup vs baseline: 1.7701x; 1.3161x over previous
import functools

import jax
import jax.numpy as jnp
from jax import lax
from jax.experimental import pallas as pl
from jax.experimental.pallas import tpu as pltpu
from jax.experimental.pallas import tpu_sc as plsc

_VOCAB = 1000000
_D = 64
_PAD = 0

_B_ROWS = 4096
_SEQ = 200
_B = _B_ROWS * _SEQ

_NC = 2
_NS = 16
_NW = _NC * _NS
_IDX_W = 128
_GPC = 4
_CHUNK = _IDX_W * _GPC
_PER_W = _B // _NW
_N_IDX_ROWS = _PER_W // _IDX_W
_N_CHUNKS = _PER_W // _CHUNK
_NBUF = 2

_NGROUPS = _B // _IDX_W
_BCOL = _B_ROWS // _IDX_W
_GPS = 32


def _gather_kernel(idx_hbm, table_hbm, out_hbm, idx_v, bufs, gsems, osems):
    wid = lax.axis_index("s") * _NC + lax.axis_index("c")
    base = wid * _PER_W
    pltpu.sync_copy(idx_hbm.at[wid], idx_v)

    def fire_gathers(t, b):
        for j in range(_GPC):
            pltpu.async_copy(
                table_hbm.at[idx_v.at[t * _GPC + j]],
                bufs[b].at[pl.ds(j * _IDX_W, _IDX_W)],
                gsems[b],
            )

    def wait_gathers(t, b):
        for j in range(_GPC):
            pltpu.make_async_copy(
                table_hbm.at[idx_v.at[t * _GPC + j]],
                bufs[b].at[pl.ds(j * _IDX_W, _IDX_W)],
                gsems[b],
            ).wait()

    def out_copy(t, b):
        return (
            bufs[b],
            out_hbm.at[pl.ds(base + t * _CHUNK, _CHUNK)],
            osems[b],
        )

    fire_gathers(0, 0)

    @pl.loop(0, _N_CHUNKS // _NBUF)
    def _(t2):
        for b in range(_NBUF):
            t = _NBUF * t2 + b
            wait_gathers(t, b)
            nb = (b + 1) % _NBUF

            @pl.when(jnp.logical_and(t >= 1, t + 1 < _N_CHUNKS))
            def _():
                pltpu.make_async_copy(*out_copy(t - 1, nb)).wait()

            @pl.when(t + 1 < _N_CHUNKS)
            def _():
                fire_gathers(t + 1, nb)

            pltpu.async_copy(*out_copy(t, b))

    pltpu.make_async_copy(*out_copy(_N_CHUNKS - 2, (_N_CHUNKS - 2) % _NBUF)).wait()
    pltpu.make_async_copy(*out_copy(_N_CHUNKS - 1, (_N_CHUNKS - 1) % _NBUF)).wait()


def _sc_gather(idx3, table):
    mesh = plsc.VectorSubcoreMesh(core_axis_name="c", subcore_axis_name="s")
    f = functools.partial(
        pl.kernel,
        out_type=jax.ShapeDtypeStruct((_B, _D), jnp.float32),
        mesh=mesh,
        scratch_types=[
            pltpu.VMEM((_N_IDX_ROWS, _IDX_W), jnp.int32),
            [pltpu.VMEM((_CHUNK, _D), jnp.float32) for _ in range(_NBUF)],
            [pltpu.SemaphoreType.DMA for _ in range(_NBUF)],
            [pltpu.SemaphoreType.DMA for _ in range(_NBUF)],
        ],
        compiler_params=pltpu.CompilerParams(use_tc_tiling_on_sc=False),
    )(_gather_kernel)
    return f(idx3, table)


def _shuffle_body(g_ref, o_ref):
    tj = lax.broadcasted_iota(jnp.int32, (_D, _IDX_W), 0)
    jj = lax.broadcasted_iota(jnp.int32, (_D, _IDX_W), 1)
    e0 = (jj == 2 * tj).astype(jnp.float32)
    e1 = (jj == 2 * tj + 1).astype(jnp.float32)
    cn = (((0,), (0,)), ((), ()))
    outs = []
    for k in range(_GPS):
        m2 = g_ref[pl.ds(k * 8, 8), :, :].reshape(_D, _IDX_W)
        a = m2[:, 0:_D]
        b = m2[:, _D:_IDX_W]
        n = lax.dot_general(a, e0, cn, preferred_element_type=jnp.float32)
        n += lax.dot_general(b, e1, cn, preferred_element_type=jnp.float32)
        outs.append(n)
    o_ref[...] = jnp.concatenate(outs, axis=1).reshape(1, _D, _GPS * _IDX_W)


def _tc_shuffle(g):
    return pl.pallas_call(
        _shuffle_body,
        grid=(_NGROUPS // _GPS,),
        in_specs=[pl.BlockSpec((8 * _GPS, 8, 128), lambda i: (i, 0, 0))],
        out_specs=pl.BlockSpec(
            (1, _D, _GPS * _IDX_W),
            lambda i: (i // (_BCOL // _GPS), 0, i % (_BCOL // _GPS)),
        ),
        out_shape=jax.ShapeDtypeStruct((_SEQ, _D, _B_ROWS), jnp.float32),
    )(g)


def _mask_body(x_ref, o_ref):
    o_ref[...] = x_ref[...] == _PAD


def _tc_mask(x):
    return pl.pallas_call(
        _mask_body,
        out_shape=jax.ShapeDtypeStruct((_B_ROWS, _SEQ), jnp.bool_),
    )(x)


def kernel(x, table):
    idx3 = (
        x.reshape(_BCOL, _IDX_W, _SEQ)
        .transpose(2, 0, 1)
        .reshape(_NW, _N_IDX_ROWS, _IDX_W)
    )
    g = _sc_gather(idx3, table).reshape(_NGROUPS * 8, 8, _IDX_W)
    hq = _tc_shuffle(g)
    h = hq.transpose(2, 0, 1)
    mask = _tc_mask(x)
    return (h, mask)
